# Initial kernel scaffold; baseline (speedup 1.0000x reference)
import jax
import jax.numpy as jnp
from jax.experimental import pallas as pl


def kernel(x, mem_kv, qkv_w, qkv_b, proj_w, proj_b):
    raise NotImplementedError("write your pallas kernel here")


# speedup vs baseline: 14.2314x; 14.2314x over previous
import functools

import jax
import jax.numpy as jnp
from jax.experimental import pallas as pl

NH = 8
HD = 64
R = 32
SCALING = HD ** (-0.5)
NEG = -3.0e38


def _qkv_kernel(x_ref, wt_ref, b_ref, g_ref, gt_ref, out_ref):
    qkv = jax.lax.dot_general(
        x_ref[...].astype(jnp.bfloat16), wt_ref[...].astype(jnp.bfloat16),
        (((1,), (0,)), ((), ())),
        preferred_element_type=jnp.float32) + b_ref[...]
    ss = jax.lax.dot_general(
        qkv * qkv, g_ref[...], (((1,), (0,)), ((), ())),
        preferred_element_type=jnp.float32,
        precision=jax.lax.Precision.HIGHEST)
    norm = jnp.maximum(jnp.sqrt(ss), 1e-12)
    seg = jax.lax.broadcasted_iota(jnp.int32, ss.shape, 1)
    f = jnp.where(seg < 2 * NH, norm, 1.0)
    e = jax.lax.dot_general(
        f, gt_ref[...], (((1,), (0,)), ((), ())),
        preferred_element_type=jnp.float32,
        precision=jax.lax.Precision.HIGHEST)
    out_ref[...] = qkv / e


def _attn_kernel(qn_ref, kn_ref, vf_ref, keyst_ref, vals_ref, out_ref):
    qn = qn_ref[...]
    kn = kn_ref[...]
    vf = vf_ref[...]
    sims = jax.lax.dot_general(
        qn.astype(jnp.bfloat16), keyst_ref[...].astype(jnp.bfloat16),
        (((1,), (0,)), ((), ())),
        preferred_element_type=jnp.float32)
    s = sims
    m0 = jnp.max(s, axis=1, keepdims=True)
    t = m0
    for _ in range(R - 1):
        s = jnp.where(s >= t, NEG, s)
        t = jnp.max(s, axis=1, keepdims=True)
    self_logit = jnp.sum(qn * kn, axis=1, keepdims=True) * SCALING
    m = jnp.maximum(m0 * SCALING, self_logit)
    w = jnp.where(sims >= t, jnp.exp(sims * SCALING - m), 0.0)
    num = jax.lax.dot_general(
        w, vals_ref[...], (((1,), (0,)), ((), ())),
        preferred_element_type=jnp.float32,
        precision=jax.lax.Precision.HIGHEST)
    den = jnp.sum(w, axis=1, keepdims=True)
    ws = jnp.exp(self_logit - m)
    out = (num + ws * vf) / (den + ws)
    out_ref[...] = out.astype(jnp.bfloat16).astype(jnp.float32)


def _proj_kernel(a_ref, wt_ref, b_ref, out_ref):
    out_ref[...] = jax.lax.dot_general(
        a_ref[...], wt_ref[...], (((1,), (0,)), ((), ())),
        preferred_element_type=jnp.float32,
        precision=jax.lax.Precision.HIGHEST) + b_ref[...]


@jax.jit
def kernel(x, mem_kv, qkv_w, qkv_b, proj_w, proj_b):
    B, D, H, W, C = x.shape
    N = D * H * W
    M = mem_kv.shape[1]
    xf = x.reshape(N, C)

    seg_of_col = jnp.arange(3 * C, dtype=jnp.int32) // HD
    g = (seg_of_col[:, None] == jnp.arange(3 * NH, dtype=jnp.int32)[None, :]
         ).astype(jnp.float32)

    qkvn = pl.pallas_call(
        _qkv_kernel,
        out_shape=jax.ShapeDtypeStruct((N, 3 * C), jnp.float32),
    )(xf, qkv_w.T, qkv_b.reshape(1, 3 * C), g, g.T)

    qkvn = qkvn.reshape(N, 3, NH, HD)
    qn = jnp.transpose(qkvn[:, 0], (1, 0, 2)).reshape(NH * N, HD)
    knf = jnp.transpose(qkvn[:, 1], (1, 0, 2)).reshape(NH * N, HD)
    vflat = jnp.transpose(qkvn[:, 2], (1, 0, 2)).reshape(NH * N, HD)

    keys_t = mem_kv[0, :, 0, :].T
    vals = mem_kv[0, :, 1, :]

    QB = 128
    grid = (NH * N // QB,)
    attn_out = pl.pallas_call(
        _attn_kernel,
        grid=grid,
        in_specs=[
            pl.BlockSpec((QB, HD), lambda i: (i, 0)),
            pl.BlockSpec((QB, HD), lambda i: (i, 0)),
            pl.BlockSpec((QB, HD), lambda i: (i, 0)),
            pl.BlockSpec((HD, M), lambda i: (0, 0)),
            pl.BlockSpec((M, HD), lambda i: (0, 0)),
        ],
        out_specs=pl.BlockSpec((QB, HD), lambda i: (i, 0)),
        out_shape=jax.ShapeDtypeStruct((NH * N, HD), jnp.float32),
    )(qn, knf, vflat, keys_t, vals)

    a = attn_out.reshape(N, C)

    y = pl.pallas_call(
        _proj_kernel,
        out_shape=jax.ShapeDtypeStruct((N, C), jnp.float32),
    )(a, proj_w.T, proj_b.reshape(1, C))

    return y.reshape(B, D, H, W, C)


# speedup vs baseline: 14.4914x; 1.0183x over previous
import functools

import jax
import jax.numpy as jnp
from jax.experimental import pallas as pl

NH = 8
HD = 64
R = 32
SCALING = HD ** (-0.5)
NEG = -3.0e38


def _qkv_kernel(x_ref, wt_ref, b_ref, g_ref, gt_ref, out_ref):
    qkv = jax.lax.dot_general(
        x_ref[...].astype(jnp.bfloat16), wt_ref[...].astype(jnp.bfloat16),
        (((1,), (0,)), ((), ())),
        preferred_element_type=jnp.float32) + b_ref[...]
    ss = jax.lax.dot_general(
        qkv * qkv, g_ref[...], (((1,), (0,)), ((), ())),
        preferred_element_type=jnp.float32,
        precision=jax.lax.Precision.HIGHEST)
    norm = jnp.maximum(jnp.sqrt(ss), 1e-12)
    seg = jax.lax.broadcasted_iota(jnp.int32, ss.shape, 1)
    f = jnp.where(seg < 2 * NH, norm, 1.0)
    e = jax.lax.dot_general(
        f, gt_ref[...], (((1,), (0,)), ((), ())),
        preferred_element_type=jnp.float32,
        precision=jax.lax.Precision.HIGHEST)
    out_ref[...] = qkv / e


def _attn_kernel(qn_ref, kn_ref, vf_ref, keyst_ref, vals_ref, out_ref):
    qn = qn_ref[...]
    kn = kn_ref[...]
    vf = vf_ref[...]
    sims = jax.lax.dot_general(
        qn.astype(jnp.bfloat16), keyst_ref[...].astype(jnp.bfloat16),
        (((1,), (0,)), ((), ())),
        preferred_element_type=jnp.float32)
    m0 = jnp.max(sims, axis=1, keepdims=True)
    t = m0
    for _ in range(R - 1):
        t = jnp.max(jnp.where(sims >= t, NEG, sims), axis=1, keepdims=True)
    self_logit = jnp.sum(qn * kn, axis=1, keepdims=True) * SCALING
    m = jnp.maximum(m0 * SCALING, self_logit)
    w = jnp.where(sims >= t, jnp.exp(sims * SCALING - m), 0.0)
    num = jax.lax.dot_general(
        w, vals_ref[...], (((1,), (0,)), ((), ())),
        preferred_element_type=jnp.float32,
        precision=jax.lax.Precision.HIGHEST)
    den = jnp.sum(w, axis=1, keepdims=True)
    ws = jnp.exp(self_logit - m)
    out = (num + ws * vf) / (den + ws)
    out_ref[...] = out.astype(jnp.bfloat16).astype(jnp.float32)


def _proj_kernel(a_ref, wt_ref, b_ref, out_ref):
    out_ref[...] = jax.lax.dot_general(
        a_ref[...], wt_ref[...], (((1,), (0,)), ((), ())),
        preferred_element_type=jnp.float32,
        precision=jax.lax.Precision.HIGHEST) + b_ref[...]


@jax.jit
def kernel(x, mem_kv, qkv_w, qkv_b, proj_w, proj_b):
    B, D, H, W, C = x.shape
    N = D * H * W
    M = mem_kv.shape[1]
    xf = x.reshape(N, C)

    seg_of_col = jnp.arange(3 * C, dtype=jnp.int32) // HD
    g = (seg_of_col[:, None] == jnp.arange(3 * NH, dtype=jnp.int32)[None, :]
         ).astype(jnp.float32)

    qkvn = pl.pallas_call(
        _qkv_kernel,
        out_shape=jax.ShapeDtypeStruct((N, 3 * C), jnp.float32),
    )(xf, qkv_w.T, qkv_b.reshape(1, 3 * C), g, g.T)

    qkvn = qkvn.reshape(N, 3, NH, HD)
    qn = jnp.transpose(qkvn[:, 0], (1, 0, 2)).reshape(NH * N, HD)
    knf = jnp.transpose(qkvn[:, 1], (1, 0, 2)).reshape(NH * N, HD)
    vflat = jnp.transpose(qkvn[:, 2], (1, 0, 2)).reshape(NH * N, HD)

    keys_t = mem_kv[0, :, 0, :].T
    vals = mem_kv[0, :, 1, :]

    QB = 128
    grid = (NH * N // QB,)
    attn_out = pl.pallas_call(
        _attn_kernel,
        grid=grid,
        in_specs=[
            pl.BlockSpec((QB, HD), lambda i: (i, 0)),
            pl.BlockSpec((QB, HD), lambda i: (i, 0)),
            pl.BlockSpec((QB, HD), lambda i: (i, 0)),
            pl.BlockSpec((HD, M), lambda i: (0, 0)),
            pl.BlockSpec((M, HD), lambda i: (0, 0)),
        ],
        out_specs=pl.BlockSpec((QB, HD), lambda i: (i, 0)),
        out_shape=jax.ShapeDtypeStruct((NH * N, HD), jnp.float32),
    )(qn, knf, vflat, keys_t, vals)

    a = attn_out.reshape(N, C)

    y = pl.pallas_call(
        _proj_kernel,
        out_shape=jax.ShapeDtypeStruct((N, C), jnp.float32),
    )(a, proj_w.T, proj_b.reshape(1, C))

    return y.reshape(B, D, H, W, C)


# speedup vs baseline: 14.5155x; 1.0017x over previous
import jax
import jax.numpy as jnp
from jax.experimental import pallas as pl

NH = 8
HD = 64
R = 32
SCALING = HD ** (-0.5)
NEG = -3.0e38


def _qkv_kernel(x_ref, wt_ref, b_ref, g_ref, gt_ref, out_ref):
    qkv = jax.lax.dot_general(
        x_ref[...].astype(jnp.bfloat16), wt_ref[...].astype(jnp.bfloat16),
        (((1,), (0,)), ((), ())),
        preferred_element_type=jnp.float32) + b_ref[...]
    ss = jax.lax.dot_general(
        qkv * qkv, g_ref[...], (((1,), (0,)), ((), ())),
        preferred_element_type=jnp.float32,
        precision=jax.lax.Precision.HIGHEST)
    norm = jnp.maximum(jnp.sqrt(ss), 1e-12)
    seg = jax.lax.broadcasted_iota(jnp.int32, ss.shape, 1)
    f = jnp.where(seg < 2 * NH, norm, 1.0)
    e = jax.lax.dot_general(
        f, gt_ref[...], (((1,), (0,)), ((), ())),
        preferred_element_type=jnp.float32,
        precision=jax.lax.Precision.HIGHEST)
    out_ref[...] = qkv / e


def _attn_kernel(qn_ref, kn_ref, vf_ref, keyst_ref, vals_ref, out_ref):
    qn = qn_ref[...]
    kn = kn_ref[...]
    vf = vf_ref[...]
    sims = jax.lax.dot_general(
        qn.astype(jnp.bfloat16), keyst_ref[...].astype(jnp.bfloat16),
        (((1,), (0,)), ((), ())),
        preferred_element_type=jnp.float32)
    m0 = jnp.max(sims, axis=1, keepdims=True)
    t = m0
    for _ in range(R - 1):
        t = jnp.max(jnp.where(sims >= t, NEG, sims), axis=1, keepdims=True)
    self_logit = jnp.sum(qn * kn, axis=1, keepdims=True) * SCALING
    m = jnp.maximum(m0 * SCALING, self_logit)
    w = jnp.where(sims >= t, jnp.exp(sims * SCALING - m), 0.0)
    num = jax.lax.dot_general(
        w, vals_ref[...], (((1,), (0,)), ((), ())),
        preferred_element_type=jnp.float32,
        precision=jax.lax.Precision.HIGHEST)
    den = jnp.sum(w, axis=1, keepdims=True)
    ws = jnp.exp(self_logit - m)
    out = (num + ws * vf) / (den + ws)
    out_ref[...] = out.astype(jnp.bfloat16).astype(jnp.float32)


def _proj_kernel(a_ref, wt_ref, b_ref, out_ref):
    out_ref[...] = jax.lax.dot_general(
        a_ref[...], wt_ref[...], (((1,), (0,)), ((), ())),
        preferred_element_type=jnp.float32,
        precision=jax.lax.Precision.HIGHEST) + b_ref[...]


@jax.jit
def kernel(x, mem_kv, qkv_w, qkv_b, proj_w, proj_b):
    B, D, H, W, C = x.shape
    N = D * H * W
    M = mem_kv.shape[1]
    xf = x.reshape(N, C)

    seg_of_col = jnp.arange(3 * C, dtype=jnp.int32) // HD
    g = (seg_of_col[:, None] == jnp.arange(3 * NH, dtype=jnp.int32)[None, :]
         ).astype(jnp.float32)

    qkvn = pl.pallas_call(
        _qkv_kernel,
        out_shape=jax.ShapeDtypeStruct((N, 3 * C), jnp.float32),
    )(xf, qkv_w.T, qkv_b.reshape(1, 3 * C), g, g.T)

    qkvn = qkvn.reshape(N, 3, NH, HD)
    qn = jnp.transpose(qkvn[:, 0], (1, 0, 2)).reshape(NH * N, HD)
    knf = jnp.transpose(qkvn[:, 1], (1, 0, 2)).reshape(NH * N, HD)
    vflat = jnp.transpose(qkvn[:, 2], (1, 0, 2)).reshape(NH * N, HD)

    keys_t = mem_kv[0, :, 0, :].T
    vals = mem_kv[0, :, 1, :]

    QB = 128
    grid = (NH * N // QB,)
    attn_out = pl.pallas_call(
        _attn_kernel,
        grid=grid,
        in_specs=[
            pl.BlockSpec((QB, HD), lambda i: (i, 0)),
            pl.BlockSpec((QB, HD), lambda i: (i, 0)),
            pl.BlockSpec((QB, HD), lambda i: (i, 0)),
            pl.BlockSpec((HD, M), lambda i: (0, 0)),
            pl.BlockSpec((M, HD), lambda i: (0, 0)),
        ],
        out_specs=pl.BlockSpec((QB, HD), lambda i: (i, 0)),
        out_shape=jax.ShapeDtypeStruct((NH * N, HD), jnp.float32),
    )(qn, knf, vflat, keys_t, vals)

    a = attn_out.reshape(N, C)

    y = pl.pallas_call(
        _proj_kernel,
        out_shape=jax.ShapeDtypeStruct((N, C), jnp.float32),
    )(a, proj_w.T, proj_b.reshape(1, C))

    return y.reshape(B, D, H, W, C)
